# Initial kernel scaffold; baseline (speedup 1.0000x reference)
import jax
import jax.numpy as jnp
from jax.experimental import pallas as pl


def kernel(rays_points_world, rays_directions_world, vertices_posed, Ts, vert_uvs, faces_unrepaired, faces_repaired, boundary_vertices):
    raise NotImplementedError("write your pallas kernel here")


# speedup vs baseline: 4.3137x; 4.3137x over previous
import functools

import jax
import jax.numpy as jnp
import numpy as np
from jax import lax
from jax.experimental import pallas as pl
from jax.experimental.pallas import tpu as pltpu

_PT_A = 64
_PT_C = 256


def _safe(x, eps=1e-12):
    return jnp.where(jnp.abs(x) < eps, eps, x)


def _closest_point(px, py, pz, ax, ay, az, bx, by, bz, cx, cy, cz):
    abx = bx - ax; aby = by - ay; abz = bz - az
    acx = cx - ax; acy = cy - ay; acz = cz - az
    apx = px - ax; apy = py - ay; apz = pz - az
    d1 = abx * apx + aby * apy + abz * apz
    d2 = acx * apx + acy * apy + acz * apz
    bpx = px - bx; bpy = py - by; bpz = pz - bz
    d3 = abx * bpx + aby * bpy + abz * bpz
    d4 = acx * bpx + acy * bpy + acz * bpz
    cpx = px - cx; cpy = py - cy; cpz = pz - cz
    d5 = abx * cpx + aby * cpy + abz * cpz
    d6 = acx * cpx + acy * cpy + acz * cpz
    va = d3 * d6 - d5 * d4
    vb = d5 * d2 - d1 * d6
    vc = d1 * d4 - d3 * d2
    v_ab = d1 / _safe(d1 - d3)
    w_ac = d2 / _safe(d2 - d6)
    w_bc = (d4 - d3) / _safe((d4 - d3) + (d5 - d6))
    denom = _safe(va + vb + vc)
    v_in = vb / denom
    w_in = vc / denom
    rx = ax + v_in * abx + w_in * acx
    ry = ay + v_in * aby + w_in * acy
    rz = az + v_in * abz + w_in * acz
    cond_bc = (va <= 0) & ((d4 - d3) >= 0) & ((d5 - d6) >= 0)
    rx = jnp.where(cond_bc, bx + w_bc * (cx - bx), rx)
    ry = jnp.where(cond_bc, by + w_bc * (cy - by), ry)
    rz = jnp.where(cond_bc, bz + w_bc * (cz - bz), rz)
    cond_ac = (vb <= 0) & (d2 >= 0) & (d6 <= 0)
    rx = jnp.where(cond_ac, ax + w_ac * acx, rx)
    ry = jnp.where(cond_ac, ay + w_ac * acy, ry)
    rz = jnp.where(cond_ac, az + w_ac * acz, rz)
    cond_ab = (vc <= 0) & (d1 >= 0) & (d3 <= 0)
    rx = jnp.where(cond_ab, ax + v_ab * abx, rx)
    ry = jnp.where(cond_ab, ay + v_ab * aby, ry)
    rz = jnp.where(cond_ab, az + v_ab * abz, rz)
    cond_c = (d6 >= 0) & (d5 <= d6)
    rx = jnp.where(cond_c, cx, rx)
    ry = jnp.where(cond_c, cy, ry)
    rz = jnp.where(cond_c, cz, rz)
    cond_b = (d3 >= 0) & (d4 <= d3)
    rx = jnp.where(cond_b, bx, rx)
    ry = jnp.where(cond_b, by, ry)
    rz = jnp.where(cond_b, bz, rz)
    cond_a = (d1 <= 0) & (d2 <= 0)
    rx = jnp.where(cond_a, ax, rx)
    ry = jnp.where(cond_a, ay, ry)
    rz = jnp.where(cond_a, az, rz)
    dx = px - rx; dy = py - ry; dz = pz - rz
    return rx, ry, rz, dx * dx + dy * dy + dz * dz


def _search_body(num_dist, pts_ref, tab_ref, out_ref):
    px = pts_ref[:, 0:1]; py = pts_ref[:, 1:2]; pz = pts_ref[:, 2:3]
    ax = tab_ref[0:1, :]; ay = tab_ref[1:2, :]; az = tab_ref[2:3, :]
    bx = tab_ref[3:4, :]; by = tab_ref[4:5, :]; bz = tab_ref[5:6, :]
    cx = tab_ref[6:7, :]; cy = tab_ref[7:8, :]; cz = tab_ref[8:9, :]
    t = tab_ref.shape[1]

    _, _, _, d2o = _closest_point(px, py, pz, ax, ay, az, bx, by, bz, cx, cy, cz)
    lane = lax.broadcasted_iota(jnp.int32, (1, t), 1)
    d2m = jnp.where(lane < num_dist, d2o, jnp.float32(jnp.inf))
    dmin = jnp.min(d2m, axis=1, keepdims=True)
    fid = jnp.min(jnp.where(d2m == dmin, lane, t), axis=1, keepdims=True)

    nax = ax - px; nay = ay - py; naz = az - pz
    nbx = bx - px; nby = by - py; nbz = bz - pz
    ncx = cx - px; ncy = cy - py; ncz = cz - pz
    la = jnp.sqrt(nax * nax + nay * nay + naz * naz + 1e-20)
    lb = jnp.sqrt(nbx * nbx + nby * nby + nbz * nbz + 1e-20)
    lc = jnp.sqrt(ncx * ncx + ncy * ncy + ncz * ncz + 1e-20)
    ux = nby * ncz - nbz * ncy
    uy = nbz * ncx - nbx * ncz
    uz = nbx * ncy - nby * ncx
    num = nax * ux + nay * uy + naz * uz
    dab = nax * nbx + nay * nby + naz * nbz
    dbc = nbx * ncx + nby * ncy + nbz * ncz
    dca = ncx * nax + ncy * nay + ncz * naz
    den = la * lb * lc + dab * lc + dbc * la + dca * lb
    den = jnp.where((jnp.abs(num) < 1e-12) & (jnp.abs(den) < 1e-12), 1e-12, den)
    omega = 2.0 * jnp.arctan2(num, den)
    wind = jnp.sum(omega, axis=1, keepdims=True) * np.float32(1.0 / (4.0 * np.pi))

    pt = out_ref.shape[0]
    out_ref[:, :] = jnp.concatenate(
        [fid.astype(jnp.float32), dmin, wind, jnp.zeros((pt, 5), jnp.float32)], axis=1)


def _uv_body(pts_ref, ares_ref, sel_ref, out_ref):
    px = pts_ref[:, 0:1]; py = pts_ref[:, 1:2]; pz = pts_ref[:, 2:3]
    dmin = ares_ref[:, 1:2]; wind = ares_ref[:, 2:3]
    ax = sel_ref[:, 0:1]; ay = sel_ref[:, 1:2]; az = sel_ref[:, 2:3]
    bx = sel_ref[:, 3:4]; by = sel_ref[:, 4:5]; bz = sel_ref[:, 5:6]
    cx = sel_ref[:, 6:7]; cy = sel_ref[:, 7:8]; cz = sel_ref[:, 8:9]
    au = sel_ref[:, 9:10]; av = sel_ref[:, 10:11]
    bu = sel_ref[:, 11:12]; bv = sel_ref[:, 12:13]
    cu = sel_ref[:, 13:14]; cv = sel_ref[:, 14:15]

    rx, ry, rz, _ = _closest_point(px, py, pz, ax, ay, az, bx, by, bz, cx, cy, cz)

    v0x = bx - ax; v0y = by - ay; v0z = bz - az
    v1x = cx - ax; v1y = cy - ay; v1z = cz - az
    v2x = rx - ax; v2y = ry - ay; v2z = rz - az
    d00 = v0x * v0x + v0y * v0y + v0z * v0z
    d01 = v0x * v1x + v0y * v1y + v0z * v1z
    d11 = v1x * v1x + v1y * v1y + v1z * v1z
    d20 = v2x * v0x + v2y * v0y + v2z * v0z
    d21 = v2x * v1x + v2y * v1y + v2z * v1z
    dn = _safe(d00 * d11 - d01 * d01)
    v = (d11 * d20 - d01 * d21) / dn
    w = (d00 * d21 - d01 * d20) / dn
    u = 1.0 - v - w
    uc = jnp.maximum(u, 0.0)
    vcl = jnp.maximum(v, 0.0)
    wc = jnp.maximum(w, 0.0)
    s = jnp.maximum(uc + vcl + wc, 1e-5)
    uc = jnp.clip(uc / s, 0.0, 1.0)
    vcl = jnp.clip(vcl / s, 0.0, 1.0)
    wc = jnp.clip(wc / s, 0.0, 1.0)
    uvu = uc * au + vcl * bu + wc * cu
    uvv = uc * av + vcl * bv + wc * cv
    sign = jnp.where(wind > 0.5, jnp.float32(-1.0), jnp.float32(1.0))
    dist = jnp.sqrt(jnp.abs(dmin) + 1e-12) * sign

    pt = out_ref.shape[0]
    out_ref[:, :] = jnp.concatenate(
        [uvu, uvv, dist, jnp.zeros((pt, 5), jnp.float32)], axis=1)


def kernel(rays_points_world, rays_directions_world, vertices_posed, Ts,
           vert_uvs, faces_unrepaired, faces_repaired, boundary_vertices):
    pts = rays_points_world.reshape(-1, 3)
    p_n = pts.shape[0]
    verts = vertices_posed.reshape(-1, 3)
    f_n = faces_unrepaired.shape[0]
    fr_n = faces_repaired.shape[0]
    t_n = ((fr_n + 127) // 128) * 128

    centroid = verts[boundary_vertices].mean(axis=0)
    verts_rep = jnp.concatenate([verts, centroid[None, :]], axis=0)

    tri_rep = verts_rep[faces_repaired]
    triuv = vert_uvs[faces_unrepaired]

    tab9 = tri_rep.transpose(1, 2, 0).reshape(9, fr_n)
    tab = jnp.zeros((16, t_n), jnp.float32).at[:9, :fr_n].set(tab9)

    sel = (jnp.zeros((t_n, 16), jnp.float32)
           .at[:fr_n, :9].set(tri_rep.reshape(fr_n, 9))
           .at[:f_n, 9:15].set(triuv.reshape(f_n, 6)))

    pts8 = jnp.zeros((p_n, 8), jnp.float32).at[:, :3].set(pts)

    ares = pl.pallas_call(
        functools.partial(_search_body, f_n),
        grid=(p_n // _PT_A,),
        in_specs=[pl.BlockSpec((_PT_A, 8), lambda i: (i, 0)),
                  pl.BlockSpec((16, t_n), lambda i: (0, 0))],
        out_specs=pl.BlockSpec((_PT_A, 8), lambda i: (i, 0)),
        out_shape=jax.ShapeDtypeStruct((p_n, 8), jnp.float32),
        compiler_params=pltpu.CompilerParams(
            dimension_semantics=("parallel",)),
    )(pts8, tab)

    fid = ares[:, 0].astype(jnp.int32)
    gath = sel[fid]

    warp8 = pl.pallas_call(
        _uv_body,
        grid=(p_n // _PT_C,),
        in_specs=[pl.BlockSpec((_PT_C, 8), lambda i: (i, 0)),
                  pl.BlockSpec((_PT_C, 8), lambda i: (i, 0)),
                  pl.BlockSpec((_PT_C, 16), lambda i: (i, 0))],
        out_specs=pl.BlockSpec((_PT_C, 8), lambda i: (i, 0)),
        out_shape=jax.ShapeDtypeStruct((p_n, 8), jnp.float32),
        compiler_params=pltpu.CompilerParams(
            dimension_semantics=("parallel",)),
    )(pts8, ares, gath)

    warped = warp8[:, :3].reshape(rays_points_world.shape)
    return (warped, rays_directions_world)


# speedup vs baseline: 4.3744x; 1.0141x over previous
import functools

import jax
import jax.numpy as jnp
import numpy as np
from jax import lax
from jax.experimental import pallas as pl
from jax.experimental.pallas import tpu as pltpu
from jax.experimental.pallas import tpu_sc as plsc

_PT_A = 64
_PT_C = 256


def _sc_gather_rows(table, idx):
    info = plsc.get_sparse_core_info()
    nc, ns = info.num_cores, info.num_subcores
    nw = nc * ns
    b = idx.shape[0]
    bw = b // nw
    mesh = plsc.VectorSubcoreMesh(core_axis_name="c", subcore_axis_name="s")

    @functools.partial(
        pl.kernel, mesh=mesh,
        out_type=jax.ShapeDtypeStruct((b, 128), jnp.float32),
        scratch_types=[pltpu.VMEM((bw,), jnp.int32),
                       pltpu.VMEM((bw, 128), jnp.float32),
                       pltpu.SemaphoreType.DMA])
    def gat(table_hbm, idx_hbm, out_hbm, idx_v, rows_v, sem):
        wid = lax.axis_index("s") * nc + lax.axis_index("c")
        base = wid * bw
        pltpu.sync_copy(idx_hbm.at[pl.ds(base, bw)], idx_v)
        pltpu.async_copy(table_hbm.at[idx_v], rows_v, sem).wait()
        pltpu.sync_copy(rows_v, out_hbm.at[pl.ds(base, bw)])

    return gat(table, idx)


def _safe(x, eps=1e-12):
    return jnp.where(jnp.abs(x) < eps, eps, x)


def _closest_point(px, py, pz, ax, ay, az, bx, by, bz, cx, cy, cz):
    abx = bx - ax; aby = by - ay; abz = bz - az
    acx = cx - ax; acy = cy - ay; acz = cz - az
    apx = px - ax; apy = py - ay; apz = pz - az
    d1 = abx * apx + aby * apy + abz * apz
    d2 = acx * apx + acy * apy + acz * apz
    bpx = px - bx; bpy = py - by; bpz = pz - bz
    d3 = abx * bpx + aby * bpy + abz * bpz
    d4 = acx * bpx + acy * bpy + acz * bpz
    cpx = px - cx; cpy = py - cy; cpz = pz - cz
    d5 = abx * cpx + aby * cpy + abz * cpz
    d6 = acx * cpx + acy * cpy + acz * cpz
    va = d3 * d6 - d5 * d4
    vb = d5 * d2 - d1 * d6
    vc = d1 * d4 - d3 * d2
    v_ab = d1 / _safe(d1 - d3)
    w_ac = d2 / _safe(d2 - d6)
    w_bc = (d4 - d3) / _safe((d4 - d3) + (d5 - d6))
    denom = _safe(va + vb + vc)
    v_in = vb / denom
    w_in = vc / denom
    rx = ax + v_in * abx + w_in * acx
    ry = ay + v_in * aby + w_in * acy
    rz = az + v_in * abz + w_in * acz
    cond_bc = (va <= 0) & ((d4 - d3) >= 0) & ((d5 - d6) >= 0)
    rx = jnp.where(cond_bc, bx + w_bc * (cx - bx), rx)
    ry = jnp.where(cond_bc, by + w_bc * (cy - by), ry)
    rz = jnp.where(cond_bc, bz + w_bc * (cz - bz), rz)
    cond_ac = (vb <= 0) & (d2 >= 0) & (d6 <= 0)
    rx = jnp.where(cond_ac, ax + w_ac * acx, rx)
    ry = jnp.where(cond_ac, ay + w_ac * acy, ry)
    rz = jnp.where(cond_ac, az + w_ac * acz, rz)
    cond_ab = (vc <= 0) & (d1 >= 0) & (d3 <= 0)
    rx = jnp.where(cond_ab, ax + v_ab * abx, rx)
    ry = jnp.where(cond_ab, ay + v_ab * aby, ry)
    rz = jnp.where(cond_ab, az + v_ab * abz, rz)
    cond_c = (d6 >= 0) & (d5 <= d6)
    rx = jnp.where(cond_c, cx, rx)
    ry = jnp.where(cond_c, cy, ry)
    rz = jnp.where(cond_c, cz, rz)
    cond_b = (d3 >= 0) & (d4 <= d3)
    rx = jnp.where(cond_b, bx, rx)
    ry = jnp.where(cond_b, by, ry)
    rz = jnp.where(cond_b, bz, rz)
    cond_a = (d1 <= 0) & (d2 <= 0)
    rx = jnp.where(cond_a, ax, rx)
    ry = jnp.where(cond_a, ay, ry)
    rz = jnp.where(cond_a, az, rz)
    dx = px - rx; dy = py - ry; dz = pz - rz
    return rx, ry, rz, dx * dx + dy * dy + dz * dz


def _search_body(num_dist, pts_ref, tab_ref, out_ref):
    px = pts_ref[:, 0:1]; py = pts_ref[:, 1:2]; pz = pts_ref[:, 2:3]
    ax = tab_ref[0:1, :]; ay = tab_ref[1:2, :]; az = tab_ref[2:3, :]
    bx = tab_ref[3:4, :]; by = tab_ref[4:5, :]; bz = tab_ref[5:6, :]
    cx = tab_ref[6:7, :]; cy = tab_ref[7:8, :]; cz = tab_ref[8:9, :]
    t = tab_ref.shape[1]

    _, _, _, d2o = _closest_point(px, py, pz, ax, ay, az, bx, by, bz, cx, cy, cz)
    lane = lax.broadcasted_iota(jnp.int32, (1, t), 1)
    d2m = jnp.where(lane < num_dist, d2o, jnp.float32(jnp.inf))
    dmin = jnp.min(d2m, axis=1, keepdims=True)
    fid = jnp.min(jnp.where(d2m == dmin, lane, t), axis=1, keepdims=True)

    nax = ax - px; nay = ay - py; naz = az - pz
    nbx = bx - px; nby = by - py; nbz = bz - pz
    ncx = cx - px; ncy = cy - py; ncz = cz - pz
    la = jnp.sqrt(nax * nax + nay * nay + naz * naz + 1e-20)
    lb = jnp.sqrt(nbx * nbx + nby * nby + nbz * nbz + 1e-20)
    lc = jnp.sqrt(ncx * ncx + ncy * ncy + ncz * ncz + 1e-20)
    ux = nby * ncz - nbz * ncy
    uy = nbz * ncx - nbx * ncz
    uz = nbx * ncy - nby * ncx
    num = nax * ux + nay * uy + naz * uz
    dab = nax * nbx + nay * nby + naz * nbz
    dbc = nbx * ncx + nby * ncy + nbz * ncz
    dca = ncx * nax + ncy * nay + ncz * naz
    den = la * lb * lc + dab * lc + dbc * la + dca * lb
    den = jnp.where((jnp.abs(num) < 1e-12) & (jnp.abs(den) < 1e-12), 1e-12, den)
    omega = 2.0 * jnp.arctan2(num, den)
    wind = jnp.sum(omega, axis=1, keepdims=True) * np.float32(1.0 / (4.0 * np.pi))

    pt = out_ref.shape[0]
    out_ref[:, :] = jnp.concatenate(
        [fid.astype(jnp.float32), dmin, wind, jnp.zeros((pt, 5), jnp.float32)], axis=1)


def _uv_body(pts_ref, ares_ref, sel_ref, out_ref):
    px = pts_ref[:, 0:1]; py = pts_ref[:, 1:2]; pz = pts_ref[:, 2:3]
    dmin = ares_ref[:, 1:2]; wind = ares_ref[:, 2:3]
    ax = sel_ref[:, 0:1]; ay = sel_ref[:, 1:2]; az = sel_ref[:, 2:3]
    bx = sel_ref[:, 3:4]; by = sel_ref[:, 4:5]; bz = sel_ref[:, 5:6]
    cx = sel_ref[:, 6:7]; cy = sel_ref[:, 7:8]; cz = sel_ref[:, 8:9]
    au = sel_ref[:, 9:10]; av = sel_ref[:, 10:11]
    bu = sel_ref[:, 11:12]; bv = sel_ref[:, 12:13]
    cu = sel_ref[:, 13:14]; cv = sel_ref[:, 14:15]

    rx, ry, rz, _ = _closest_point(px, py, pz, ax, ay, az, bx, by, bz, cx, cy, cz)

    v0x = bx - ax; v0y = by - ay; v0z = bz - az
    v1x = cx - ax; v1y = cy - ay; v1z = cz - az
    v2x = rx - ax; v2y = ry - ay; v2z = rz - az
    d00 = v0x * v0x + v0y * v0y + v0z * v0z
    d01 = v0x * v1x + v0y * v1y + v0z * v1z
    d11 = v1x * v1x + v1y * v1y + v1z * v1z
    d20 = v2x * v0x + v2y * v0y + v2z * v0z
    d21 = v2x * v1x + v2y * v1y + v2z * v1z
    dn = _safe(d00 * d11 - d01 * d01)
    v = (d11 * d20 - d01 * d21) / dn
    w = (d00 * d21 - d01 * d20) / dn
    u = 1.0 - v - w
    uc = jnp.maximum(u, 0.0)
    vcl = jnp.maximum(v, 0.0)
    wc = jnp.maximum(w, 0.0)
    s = jnp.maximum(uc + vcl + wc, 1e-5)
    uc = jnp.clip(uc / s, 0.0, 1.0)
    vcl = jnp.clip(vcl / s, 0.0, 1.0)
    wc = jnp.clip(wc / s, 0.0, 1.0)
    uvu = uc * au + vcl * bu + wc * cu
    uvv = uc * av + vcl * bv + wc * cv
    sign = jnp.where(wind > 0.5, jnp.float32(-1.0), jnp.float32(1.0))
    dist = jnp.sqrt(jnp.abs(dmin) + 1e-12) * sign

    pt = out_ref.shape[0]
    out_ref[:, :] = jnp.concatenate(
        [uvu, uvv, dist, jnp.zeros((pt, 5), jnp.float32)], axis=1)


def kernel(rays_points_world, rays_directions_world, vertices_posed, Ts,
           vert_uvs, faces_unrepaired, faces_repaired, boundary_vertices):
    pts = rays_points_world.reshape(-1, 3)
    p_n = pts.shape[0]
    verts = vertices_posed.reshape(-1, 3)
    f_n = faces_unrepaired.shape[0]
    fr_n = faces_repaired.shape[0]
    t_n = ((fr_n + 127) // 128) * 128

    centroid = verts[boundary_vertices].mean(axis=0)
    verts_rep = jnp.concatenate([verts, centroid[None, :]], axis=0)
    v_n = verts_rep.shape[0]

    table = jnp.zeros((v_n + verts.shape[0], 128), jnp.float32)
    table = table.at[:v_n, :3].set(verts_rep)
    table = table.at[v_n:, :2].set(vert_uvs)
    flat_idx = jnp.concatenate(
        [faces_repaired.reshape(-1), faces_unrepaired.reshape(-1) + v_n])
    ni = flat_idx.shape[0]
    ni_pad = ((ni + 255) // 256) * 256
    flat_idx = jnp.zeros((ni_pad,), jnp.int32).at[:ni].set(flat_idx)
    grows = _sc_gather_rows(table, flat_idx)
    tri_rep = grows[:3 * fr_n, :3].reshape(fr_n, 3, 3)
    triuv = grows[3 * fr_n:ni, :2].reshape(f_n, 3, 2)

    tab9 = tri_rep.transpose(1, 2, 0).reshape(9, fr_n)
    tab = jnp.zeros((16, t_n), jnp.float32).at[:9, :fr_n].set(tab9)

    sel = (jnp.zeros((t_n, 128), jnp.float32)
           .at[:fr_n, :9].set(tri_rep.reshape(fr_n, 9))
           .at[:f_n, 9:15].set(triuv.reshape(f_n, 6)))

    pts8 = jnp.zeros((p_n, 8), jnp.float32).at[:, :3].set(pts)

    ares = pl.pallas_call(
        functools.partial(_search_body, f_n),
        grid=(p_n // _PT_A,),
        in_specs=[pl.BlockSpec((_PT_A, 8), lambda i: (i, 0)),
                  pl.BlockSpec((16, t_n), lambda i: (0, 0))],
        out_specs=pl.BlockSpec((_PT_A, 8), lambda i: (i, 0)),
        out_shape=jax.ShapeDtypeStruct((p_n, 8), jnp.float32),
        compiler_params=pltpu.CompilerParams(
            dimension_semantics=("parallel",)),
    )(pts8, tab)

    fid = ares[:, 0].astype(jnp.int32)
    gath = _sc_gather_rows(sel, fid)

    warp8 = pl.pallas_call(
        _uv_body,
        grid=(p_n // _PT_C,),
        in_specs=[pl.BlockSpec((_PT_C, 8), lambda i: (i, 0)),
                  pl.BlockSpec((_PT_C, 8), lambda i: (i, 0)),
                  pl.BlockSpec((_PT_C, 128), lambda i: (i, 0))],
        out_specs=pl.BlockSpec((_PT_C, 8), lambda i: (i, 0)),
        out_shape=jax.ShapeDtypeStruct((p_n, 8), jnp.float32),
        compiler_params=pltpu.CompilerParams(
            dimension_semantics=("parallel",)),
    )(pts8, ares, gath)

    warped = warp8[:, :3].reshape(rays_points_world.shape)
    return (warped, rays_directions_world)


# speedup vs baseline: 6.6939x; 1.5303x over previous
import functools

import jax
import jax.numpy as jnp
import numpy as np
from jax import lax
from jax.experimental import pallas as pl
from jax.experimental.pallas import tpu as pltpu
from jax.experimental.pallas import tpu_sc as plsc

_PT_A = 64


def _sc_gather_rows(table, idx):
    info = plsc.get_sparse_core_info()
    nc, ns = info.num_cores, info.num_subcores
    nw = nc * ns
    b = idx.shape[0]
    bw = b // nw
    mesh = plsc.VectorSubcoreMesh(core_axis_name="c", subcore_axis_name="s")

    @functools.partial(
        pl.kernel, mesh=mesh,
        out_type=jax.ShapeDtypeStruct((b, 128), jnp.float32),
        scratch_types=[pltpu.VMEM((bw,), jnp.int32),
                       pltpu.VMEM((bw, 128), jnp.float32),
                       pltpu.SemaphoreType.DMA])
    def gat(table_hbm, idx_hbm, out_hbm, idx_v, rows_v, sem):
        wid = lax.axis_index("s") * nc + lax.axis_index("c")
        base = wid * bw
        pltpu.sync_copy(idx_hbm.at[pl.ds(base, bw)], idx_v)
        pltpu.async_copy(table_hbm.at[idx_v], rows_v, sem).wait()
        pltpu.sync_copy(rows_v, out_hbm.at[pl.ds(base, bw)])

    return gat(table, idx)


def _safe(x, eps=1e-12):
    return jnp.where(jnp.abs(x) < eps, eps, x)


def _search_body(num_dist, pts_ref, tab_ref, out_ref):
    px = pts_ref[:, 0:1]; py = pts_ref[:, 1:2]; pz = pts_ref[:, 2:3]
    t = tab_ref.shape[1]
    cw = 256
    pt = out_ref.shape[0]

    best_d = jnp.full((pt, 1), jnp.inf, jnp.float32)
    best_i = jnp.full((pt, 1), t, jnp.int32)
    wind_acc = jnp.zeros((pt, 1), jnp.float32)

    for k in range(t // cw):
        sl = pl.ds(k * cw, cw)
        ax = tab_ref[0:1, sl]; ay = tab_ref[1:2, sl]; az = tab_ref[2:3, sl]
        bx = tab_ref[3:4, sl]; by = tab_ref[4:5, sl]; bz = tab_ref[5:6, sl]
        cx = tab_ref[6:7, sl]; cy = tab_ref[7:8, sl]; cz = tab_ref[8:9, sl]

        abx = bx - ax; aby = by - ay; abz = bz - az
        acx = cx - ax; acy = cy - ay; acz = cz - az
        r00 = 1.0 / _safe(abx * abx + aby * aby + abz * abz)
        r11 = 1.0 / _safe(acx * acx + acy * acy + acz * acz)
        bcx = cx - bx; bcy = cy - by; bcz = cz - bz
        rbc = 1.0 / _safe(bcx * bcx + bcy * bcy + bcz * bcz)

        apx = px - ax; apy = py - ay; apz = pz - az
        bpx = px - bx; bpy = py - by; bpz = pz - bz
        cpx = px - cx; cpy = py - cy; cpz = pz - cz
        ap2 = apx * apx + apy * apy + apz * apz
        bp2 = bpx * bpx + bpy * bpy + bpz * bpz
        cp2 = cpx * cpx + cpy * cpy + cpz * cpz

        d1 = abx * apx + aby * apy + abz * apz
        d2 = acx * apx + acy * apy + acz * apz
        d3 = abx * bpx + aby * bpy + abz * bpz
        d4 = acx * bpx + acy * bpy + acz * bpz
        d5 = abx * cpx + aby * cpy + abz * cpz
        d6 = acx * cpx + acy * cpy + acz * cpz
        va = d3 * d6 - d5 * d4
        vb = d5 * d2 - d1 * d6
        vc = d1 * d4 - d3 * d2
        d43 = d4 - d3
        d56 = d5 - d6
        denom = _safe(va + vb + vc)
        rden = 1.0 / denom

        one = jnp.float32(1.0)
        zero = jnp.float32(0.0)
        v = vb * rden
        w = vc * rden
        w_bc = d43 * rbc
        cond = (va <= 0) & (d43 >= 0) & (d56 >= 0)
        v = jnp.where(cond, one - w_bc, v)
        w = jnp.where(cond, w_bc, w)
        cond = (vb <= 0) & (d2 >= 0) & (d6 <= 0)
        v = jnp.where(cond, zero, v)
        w = jnp.where(cond, d2 * r11, w)
        cond = (vc <= 0) & (d1 >= 0) & (d3 <= 0)
        v = jnp.where(cond, d1 * r00, v)
        w = jnp.where(cond, zero, w)
        cond = (d6 >= 0) & (d5 <= d6)
        v = jnp.where(cond, zero, v)
        w = jnp.where(cond, one, w)
        cond = (d3 >= 0) & (d4 <= d3)
        v = jnp.where(cond, one, v)
        w = jnp.where(cond, zero, w)
        cond = (d1 <= 0) & (d2 <= 0)
        v = jnp.where(cond, zero, v)
        w = jnp.where(cond, zero, w)
        dx = apx - v * abx - w * acx
        dy = apy - v * aby - w * acy
        dz = apz - v * abz - w * acz
        d2o = dx * dx + dy * dy + dz * dz

        lane = lax.broadcasted_iota(jnp.int32, (1, cw), 1) + (k * cw)
        d2m = jnp.where(lane < num_dist, d2o, jnp.float32(jnp.inf))
        dmin_c = jnp.min(d2m, axis=1, keepdims=True)
        fid_c = jnp.min(jnp.where(d2m == dmin_c, lane, t), axis=1,
                        keepdims=True)
        upd = dmin_c < best_d
        best_i = jnp.where(upd, fid_c, best_i)
        best_d = jnp.where(upd, dmin_c, best_d)

        la = jnp.sqrt(ap2 + 1e-20)
        lb = jnp.sqrt(bp2 + 1e-20)
        lc = jnp.sqrt(cp2 + 1e-20)
        ux = bpy * cpz - bpz * cpy
        uy = bpz * cpx - bpx * cpz
        uz = bpx * cpy - bpy * cpx
        num = -(apx * ux + apy * uy + apz * uz)
        dab = apx * bpx + apy * bpy + apz * bpz
        dbc = bpx * cpx + bpy * cpy + bpz * cpz
        dca = cpx * apx + cpy * apy + cpz * apz
        den = la * lb * lc + dab * lc + dbc * la + dca * lb
        den = jnp.where((jnp.abs(num) < 1e-12) & (jnp.abs(den) < 1e-12),
                        1e-12, den)
        omega = jnp.arctan2(num, den)
        wind_acc = wind_acc + jnp.sum(omega, axis=1, keepdims=True)

    dmin = best_d
    fid = best_i
    wind = wind_acc * np.float32(2.0 / (4.0 * np.pi))

    sign = jnp.where(wind > 0.5, jnp.float32(-1.0), jnp.float32(1.0))
    dist = jnp.sqrt(jnp.abs(dmin) + 1e-12) * sign

    pt = out_ref.shape[0]
    out_ref[:, :] = jnp.concatenate(
        [fid.astype(jnp.float32), dist, jnp.zeros((pt, 6), jnp.float32)], axis=1)


def _uv_body(sin_ref, out_ref):
    def plane(k):
        return sin_ref[8 * k:8 * (k + 1), :]
    px, py, pz = plane(0), plane(1), plane(2)
    ax, ay, az = plane(3), plane(4), plane(5)
    bx, by, bz = plane(6), plane(7), plane(8)
    cx, cy, cz = plane(9), plane(10), plane(11)
    au, av = plane(12), plane(13)
    bu, bv = plane(14), plane(15)
    cu, cv = plane(16), plane(17)

    abx = bx - ax; aby = by - ay; abz = bz - az
    acx = cx - ax; acy = cy - ay; acz = cz - az
    apx = px - ax; apy = py - ay; apz = pz - az
    d1 = abx * apx + aby * apy + abz * apz
    d2 = acx * apx + acy * apy + acz * apz
    bpx = px - bx; bpy = py - by; bpz = pz - bz
    d3 = abx * bpx + aby * bpy + abz * bpz
    d4 = acx * bpx + acy * bpy + acz * bpz
    cpx = px - cx; cpy = py - cy; cpz = pz - cz
    d5 = abx * cpx + aby * cpy + abz * cpz
    d6 = acx * cpx + acy * cpy + acz * cpz
    va = d3 * d6 - d5 * d4
    vb = d5 * d2 - d1 * d6
    vc = d1 * d4 - d3 * d2
    v_ab = d1 / _safe(d1 - d3)
    w_ac = d2 / _safe(d2 - d6)
    w_bc = (d4 - d3) / _safe((d4 - d3) + (d5 - d6))
    denom = _safe(va + vb + vc)
    v_in = vb / denom
    w_in = vc / denom
    rx = ax + v_in * abx + w_in * acx
    ry = ay + v_in * aby + w_in * acy
    rz = az + v_in * abz + w_in * acz
    cond_bc = (va <= 0) & ((d4 - d3) >= 0) & ((d5 - d6) >= 0)
    rx = jnp.where(cond_bc, bx + w_bc * (cx - bx), rx)
    ry = jnp.where(cond_bc, by + w_bc * (cy - by), ry)
    rz = jnp.where(cond_bc, bz + w_bc * (cz - bz), rz)
    cond_ac = (vb <= 0) & (d2 >= 0) & (d6 <= 0)
    rx = jnp.where(cond_ac, ax + w_ac * acx, rx)
    ry = jnp.where(cond_ac, ay + w_ac * acy, ry)
    rz = jnp.where(cond_ac, az + w_ac * acz, rz)
    cond_ab = (vc <= 0) & (d1 >= 0) & (d3 <= 0)
    rx = jnp.where(cond_ab, ax + v_ab * abx, rx)
    ry = jnp.where(cond_ab, ay + v_ab * aby, ry)
    rz = jnp.where(cond_ab, az + v_ab * abz, rz)
    cond_c = (d6 >= 0) & (d5 <= d6)
    rx = jnp.where(cond_c, cx, rx)
    ry = jnp.where(cond_c, cy, ry)
    rz = jnp.where(cond_c, cz, rz)
    cond_b = (d3 >= 0) & (d4 <= d3)
    rx = jnp.where(cond_b, bx, rx)
    ry = jnp.where(cond_b, by, ry)
    rz = jnp.where(cond_b, bz, rz)
    cond_a = (d1 <= 0) & (d2 <= 0)
    rx = jnp.where(cond_a, ax, rx)
    ry = jnp.where(cond_a, ay, ry)
    rz = jnp.where(cond_a, az, rz)

    v0x = abx; v0y = aby; v0z = abz
    v1x = acx; v1y = acy; v1z = acz
    v2x = rx - ax; v2y = ry - ay; v2z = rz - az
    d00 = v0x * v0x + v0y * v0y + v0z * v0z
    d01 = v0x * v1x + v0y * v1y + v0z * v1z
    d11 = v1x * v1x + v1y * v1y + v1z * v1z
    d20 = v2x * v0x + v2y * v0y + v2z * v0z
    d21 = v2x * v1x + v2y * v1y + v2z * v1z
    dn = _safe(d00 * d11 - d01 * d01)
    v = (d11 * d20 - d01 * d21) / dn
    w = (d00 * d21 - d01 * d20) / dn
    u = 1.0 - v - w
    uc = jnp.maximum(u, 0.0)
    vcl = jnp.maximum(v, 0.0)
    wc = jnp.maximum(w, 0.0)
    s = jnp.maximum(uc + vcl + wc, 1e-5)
    uc = jnp.clip(uc / s, 0.0, 1.0)
    vcl = jnp.clip(vcl / s, 0.0, 1.0)
    wc = jnp.clip(wc / s, 0.0, 1.0)
    uvu = uc * au + vcl * bu + wc * cu
    uvv = uc * av + vcl * bv + wc * cv
    out_ref[:, :] = jnp.concatenate([uvu, uvv], axis=0)


def kernel(rays_points_world, rays_directions_world, vertices_posed, Ts,
           vert_uvs, faces_unrepaired, faces_repaired, boundary_vertices):
    pts = rays_points_world.reshape(-1, 3)
    p_n = pts.shape[0]
    verts = vertices_posed.reshape(-1, 3)
    f_n = faces_unrepaired.shape[0]
    fr_n = faces_repaired.shape[0]
    t_n = ((fr_n + 127) // 128) * 128

    centroid = verts[boundary_vertices].mean(axis=0)
    verts_rep = jnp.concatenate([verts, centroid[None, :]], axis=0)
    v_n = verts_rep.shape[0]

    table = jnp.zeros((v_n + verts.shape[0], 128), jnp.float32)
    table = table.at[:v_n, :3].set(verts_rep)
    table = table.at[v_n:, :2].set(vert_uvs)
    flat_idx = jnp.concatenate(
        [faces_repaired.reshape(-1), faces_unrepaired.reshape(-1) + v_n])
    ni = flat_idx.shape[0]
    ni_pad = ((ni + 255) // 256) * 256
    flat_idx = jnp.zeros((ni_pad,), jnp.int32).at[:ni].set(flat_idx)
    grows = _sc_gather_rows(table, flat_idx)
    tri_rep = grows[:3 * fr_n, :3].reshape(fr_n, 3, 3)
    triuv = grows[3 * fr_n:ni, :2].reshape(f_n, 3, 2)

    tab9 = tri_rep.transpose(1, 2, 0).reshape(9, fr_n)
    tab = jnp.zeros((16, t_n), jnp.float32).at[:9, :fr_n].set(tab9)

    sel = (jnp.zeros((t_n, 128), jnp.float32)
           .at[:fr_n, :9].set(tri_rep.reshape(fr_n, 9))
           .at[:f_n, 9:15].set(triuv.reshape(f_n, 6)))

    pts8 = jnp.zeros((p_n, 8), jnp.float32).at[:, :3].set(pts)

    ares = pl.pallas_call(
        functools.partial(_search_body, f_n),
        grid=(p_n // _PT_A,),
        in_specs=[pl.BlockSpec((_PT_A, 8), lambda i: (i, 0)),
                  pl.BlockSpec((16, t_n), lambda i: (0, 0))],
        out_specs=pl.BlockSpec((_PT_A, 8), lambda i: (i, 0)),
        out_shape=jax.ShapeDtypeStruct((p_n, 8), jnp.float32),
        compiler_params=pltpu.CompilerParams(
            dimension_semantics=("parallel",)),
    )(pts8, tab)

    fid = ares[:, 0].astype(jnp.int32)
    dist = ares[:, 1]
    gath = _sc_gather_rows(sel, fid)

    w_n = p_n // 8
    planes = [pts[:, 0], pts[:, 1], pts[:, 2]] + [gath[:, k] for k in range(15)]
    sin = jnp.concatenate([x.reshape(8, w_n) for x in planes], axis=0)

    uv16 = pl.pallas_call(
        _uv_body,
        grid=(1,),
        in_specs=[pl.BlockSpec((144, w_n), lambda i: (0, 0))],
        out_specs=pl.BlockSpec((16, w_n), lambda i: (0, 0)),
        out_shape=jax.ShapeDtypeStruct((16, w_n), jnp.float32),
    )(sin)

    uvu = uv16[0:8, :].reshape(-1)
    uvv = uv16[8:16, :].reshape(-1)
    warped = jnp.stack([uvu, uvv, dist], axis=1).reshape(rays_points_world.shape)
    return (warped, rays_directions_world)
